# baseline (speedup 1.0000x reference)
import jax
import jax.numpy as jnp
from jax import lax
from jax.experimental import pallas as pl
from jax.experimental.pallas import tpu as pltpu
from jax.experimental.pallas import tpu_sc as plsc

N_NODES = 10000
E_EDGES = 320000
D_IN = 128
U_HID = 16

NC = 2
NS = 16
N_TILES = NC * NS
CHUNK = 128
CHUNKS_PER_TILE = 80
EDGES_PER_TILE = CHUNKS_PER_TILE * CHUNK
E_PAD = N_TILES * EDGES_PER_TILE
N_PAD = 10240
ROWS_PER_TILE = N_PAD // NS
NBUF = 4
OVERRUN = NBUF * CHUNK
LAST_REAL = E_EDGES - (N_TILES - 1) * EDGES_PER_TILE
PAD_E_TILE = EDGES_PER_TILE - LAST_REAL
CHUNKS_SRC = CHUNKS_PER_TILE + NBUF


def _load_idx(flat_hbm, pad_hbm, idx, tile):
    base = tile * EDGES_PER_TILE

    @pl.when(tile < N_TILES - 1)
    def _():
        pltpu.sync_copy(flat_hbm.at[pl.ds(base, EDGES_PER_TILE)],
                        idx.at[pl.ds(0, EDGES_PER_TILE)])

    @pl.when(tile == N_TILES - 1)
    def _():
        pltpu.sync_copy(flat_hbm.at[pl.ds(base, LAST_REAL)],
                        idx.at[pl.ds(0, LAST_REAL)])
        pltpu.sync_copy(pad_hbm.at[pl.ds(0, PAD_E_TILE)],
                        idx.at[pl.ds(LAST_REAL, PAD_E_TILE)])


def _deg_body(dst_hbm, pad_hbm, ones_hbm, zeros_hbm, out_hbm,
              acc, didx, ones_v, sem):
    cid = lax.axis_index("c")
    sid = lax.axis_index("s")
    row0 = sid * ROWS_PER_TILE
    rows = pl.ds(row0, ROWS_PER_TILE)
    pltpu.sync_copy(zeros_hbm.at[rows], acc.at[rows])
    pltpu.sync_copy(ones_hbm, ones_v)
    tile = cid * NS + sid
    _load_idx(dst_hbm, pad_hbm, didx, tile)
    plsc.subcore_barrier()

    @pl.loop(0, CHUNKS_PER_TILE)
    def _(k):
        pltpu.async_copy(ones_v, acc.at[didx.at[pl.ds(k * CHUNK, CHUNK)]],
                         sem, add=True)

    @pl.loop(0, CHUNKS_PER_TILE)
    def _(k):
        pltpu.make_async_copy(ones_hbm, ones_v, sem).wait()

    plsc.subcore_barrier()
    pltpu.sync_copy(acc.at[rows], out_hbm.at[cid].at[rows])


def _scat_body(g_hbm, src_hbm, dst_hbm, pad_hbm, zeros_hbm, out_hbm,
               acc, gv, sidx, didx, bufs, gsems, ssems):
    cid = lax.axis_index("c")
    sid = lax.axis_index("s")
    row0 = sid * ROWS_PER_TILE
    rows = pl.ds(row0, ROWS_PER_TILE)
    pltpu.sync_copy(zeros_hbm.at[rows], acc.at[rows])
    pltpu.sync_copy(g_hbm.at[rows], gv.at[rows])
    tile = cid * NS + sid
    _load_idx(src_hbm, pad_hbm, sidx, tile)
    pltpu.sync_copy(pad_hbm.at[pl.ds(0, OVERRUN)],
                    sidx.at[pl.ds(EDGES_PER_TILE, OVERRUN)])
    _load_idx(dst_hbm, pad_hbm, didx, tile)
    plsc.subcore_barrier()

    def s_at(c):
        return sidx.at[pl.ds(c * CHUNK, CHUNK)]

    def d_at(c):
        return didx.at[pl.ds(c * CHUNK, CHUNK)]

    for b in range(NBUF):
        pltpu.async_copy(gv.at[s_at(b)], bufs.at[b], gsems.at[b])

    @pl.loop(0, CHUNKS_PER_TILE, step=NBUF)
    def _(k):
        for b in range(NBUF):
            pltpu.make_async_copy(gv.at[s_at(b)], bufs.at[b],
                                  gsems.at[b]).wait()
            pltpu.async_copy(bufs.at[b], acc.at[d_at(k + b)], ssems.at[b],
                             add=True)
        for b in range(NBUF):
            pltpu.make_async_copy(bufs.at[b], acc.at[pl.ds(0, CHUNK)],
                                  ssems.at[b]).wait()
            pltpu.async_copy(gv.at[s_at(k + b + NBUF)], bufs.at[b],
                             gsems.at[b])

    for b in range(NBUF):
        pltpu.make_async_copy(gv.at[s_at(b)], bufs.at[b], gsems.at[b]).wait()

    plsc.subcore_barrier()
    pltpu.sync_copy(acc.at[rows], out_hbm.at[cid].at[rows])


_sc_calls_cache = []


def _sc_calls():
    if not _sc_calls_cache:
        mesh = plsc.VectorSubcoreMesh(core_axis_name="c", subcore_axis_name="s",
                                      num_cores=NC, num_subcores=NS)
        cp = pltpu.CompilerParams(use_tc_tiling_on_sc=False)
        deg_call = pl.kernel(
            _deg_body,
            out_type=jax.ShapeDtypeStruct((NC, N_PAD, U_HID), jnp.float32),
            mesh=mesh,
            scratch_types=[
                pltpu.VMEM_SHARED((N_PAD, U_HID), jnp.float32),
                pltpu.VMEM((EDGES_PER_TILE,), jnp.int32),
                pltpu.VMEM((CHUNK, U_HID), jnp.float32),
                pltpu.SemaphoreType.DMA,
            ],
            compiler_params=cp,
        )
        scat_call = pl.kernel(
            _scat_body,
            out_type=jax.ShapeDtypeStruct((NC, N_PAD, U_HID), jnp.float32),
            mesh=mesh,
            scratch_types=[
                pltpu.VMEM_SHARED((N_PAD, U_HID), jnp.float32),
                pltpu.VMEM_SHARED((N_PAD, U_HID), jnp.float32),
                pltpu.VMEM((CHUNKS_SRC * CHUNK,), jnp.int32),
                pltpu.VMEM((EDGES_PER_TILE,), jnp.int32),
                pltpu.VMEM((NBUF, CHUNK, U_HID), jnp.float32),
                pltpu.SemaphoreType.DMA((NBUF,)),
                pltpu.SemaphoreType.DMA((NBUF,)),
            ],
            compiler_params=cp,
        )
        _sc_calls_cache.append((deg_call, scat_call))
    return _sc_calls_cache[0]


def _combine1_body(p_ref, x_ref, w_ref, dinv_ref, g_ref):
    deg = p_ref[0] + p_ref[1] + 1.0
    dinv = lax.rsqrt(jnp.maximum(deg, 1e-12))
    dinv_ref[...] = dinv
    hw = jnp.dot(x_ref[...], w_ref[...], preferred_element_type=jnp.float32)
    g_ref[...] = dinv * hw


def _layer2_body(p_ref, g1_ref, dinv_ref, w2_ref, b1_ref, g2_ref):
    dinv = dinv_ref[...]
    h1 = jnp.maximum(dinv * (p_ref[0] + p_ref[1] + g1_ref[...]) + b1_ref[...],
                     0.0)
    hw2 = jnp.dot(h1, w2_ref[...], preferred_element_type=jnp.float32)
    g2_ref[...] = dinv * hw2


def _final_body(p_ref, g2_ref, dinv_ref, b2_ref, o_ref):
    o_ref[...] = jnp.maximum(
        dinv_ref[...] * (p_ref[0] + p_ref[1] + g2_ref[...]) + b2_ref[...], 0.0)


def kernel(x, edge_index, W1, b1, W2, b2):
    src = edge_index[0]
    dst = edge_index[1]
    pad_idx = jnp.full((PAD_E_TILE,), N_NODES, dtype=jnp.int32)
    x_pad = jnp.pad(x, ((0, N_PAD - N_NODES), (0, 0)))
    zeros = jnp.zeros((N_PAD, U_HID), jnp.float32)
    ones128 = jnp.ones((CHUNK, U_HID), jnp.float32)
    b1r = b1.reshape(1, U_HID)
    b2r = b2.reshape(1, U_HID)

    f32 = jnp.float32
    nu = jax.ShapeDtypeStruct((N_PAD, U_HID), f32)
    _deg_call, _scat_call = _sc_calls()

    deg_part = _deg_call(dst, pad_idx, ones128, zeros)
    dinv, g1 = pl.pallas_call(_combine1_body, out_shape=(nu, nu))(
        deg_part, x_pad, W1)
    part1 = _scat_call(g1, src, dst, pad_idx, zeros)
    g2 = pl.pallas_call(_layer2_body, out_shape=nu)(
        part1, g1, dinv, W2, b1r)
    part2 = _scat_call(g2, src, dst, pad_idx, zeros)
    out = pl.pallas_call(_final_body, out_shape=nu)(part2, g2, dinv, b2r)
    return out[:N_NODES]


# speedup vs baseline: 1.0713x; 1.0713x over previous
import jax
import jax.numpy as jnp
from jax import lax
from jax.experimental import pallas as pl
from jax.experimental.pallas import tpu as pltpu
from jax.experimental.pallas import tpu_sc as plsc

N_NODES = 10000
E_EDGES = 320000
D_IN = 128
U_HID = 16

NC = 2
NS = 16
N_TILES = NC * NS
CHUNK = 128
CHUNKS_PER_TILE = 80
EDGES_PER_TILE = CHUNKS_PER_TILE * CHUNK
E_PAD = N_TILES * EDGES_PER_TILE
N_PAD = 10240
ROWS_PER_TILE = N_PAD // NS
NBUF = 4
OVERRUN = NBUF * CHUNK
LAST_REAL = E_EDGES - (N_TILES - 1) * EDGES_PER_TILE
PAD_E_TILE = EDGES_PER_TILE - LAST_REAL
CHUNKS_SRC = CHUNKS_PER_TILE + NBUF


def _load_idx(flat_hbm, pad_hbm, idx, tile, sem):
    base = tile * EDGES_PER_TILE

    @pl.when(tile < N_TILES - 1)
    def _():
        pltpu.async_copy(flat_hbm.at[pl.ds(base, EDGES_PER_TILE)],
                         idx.at[pl.ds(0, EDGES_PER_TILE)], sem)

    @pl.when(tile == N_TILES - 1)
    def _():
        pltpu.async_copy(flat_hbm.at[pl.ds(base, LAST_REAL)],
                         idx.at[pl.ds(0, LAST_REAL)], sem)
        pltpu.async_copy(pad_hbm.at[pl.ds(0, PAD_E_TILE)],
                         idx.at[pl.ds(LAST_REAL, PAD_E_TILE)], sem)


def _wait_idx(flat_hbm, idx, sem):
    pltpu.make_async_copy(flat_hbm.at[pl.ds(0, EDGES_PER_TILE)],
                          idx.at[pl.ds(0, EDGES_PER_TILE)], sem).wait()


def _deg_body(dst_hbm, pad_hbm, ones_hbm, zeros_hbm, out_hbm,
              acc, didx, ones_v, sem):
    cid = lax.axis_index("c")
    sid = lax.axis_index("s")
    row0 = sid * ROWS_PER_TILE
    rows = pl.ds(row0, ROWS_PER_TILE)
    pltpu.async_copy(zeros_hbm.at[rows], acc.at[rows], sem)
    pltpu.async_copy(ones_hbm, ones_v, sem)
    tile = cid * NS + sid
    _load_idx(dst_hbm, pad_hbm, didx, tile, sem)
    pltpu.make_async_copy(zeros_hbm.at[rows], acc.at[rows], sem).wait()
    pltpu.make_async_copy(ones_hbm, ones_v, sem).wait()
    _wait_idx(dst_hbm, didx, sem)
    plsc.subcore_barrier()

    @pl.loop(0, CHUNKS_PER_TILE)
    def _(k):
        pltpu.async_copy(ones_v, acc.at[didx.at[pl.ds(k * CHUNK, CHUNK)]],
                         sem, add=True)

    @pl.loop(0, CHUNKS_PER_TILE)
    def _(k):
        pltpu.make_async_copy(ones_hbm, ones_v, sem).wait()

    plsc.subcore_barrier()
    pltpu.sync_copy(acc.at[rows], out_hbm.at[cid].at[rows])


def _scat_body(g_hbm, src_hbm, dst_hbm, pad_hbm, zeros_hbm, out_hbm,
               acc, gv, sidx, didx, bufs, gsems, ssem):
    cid = lax.axis_index("c")
    sid = lax.axis_index("s")
    row0 = sid * ROWS_PER_TILE
    rows = pl.ds(row0, ROWS_PER_TILE)
    pltpu.async_copy(zeros_hbm.at[rows], acc.at[rows], ssem)
    pltpu.async_copy(g_hbm.at[rows], gv.at[rows], ssem)
    tile = cid * NS + sid
    _load_idx(src_hbm, pad_hbm, sidx, tile, ssem)
    pltpu.async_copy(pad_hbm.at[pl.ds(0, OVERRUN)],
                     sidx.at[pl.ds(EDGES_PER_TILE, OVERRUN)], ssem)
    _load_idx(dst_hbm, pad_hbm, didx, tile, ssem)
    pltpu.make_async_copy(zeros_hbm.at[rows], acc.at[rows], ssem).wait()
    pltpu.make_async_copy(g_hbm.at[rows], gv.at[rows], ssem).wait()
    _wait_idx(src_hbm, sidx, ssem)
    pltpu.make_async_copy(pad_hbm.at[pl.ds(0, OVERRUN)],
                          sidx.at[pl.ds(EDGES_PER_TILE, OVERRUN)],
                          ssem).wait()
    _wait_idx(dst_hbm, didx, ssem)
    plsc.subcore_barrier()

    def s_at(c):
        return sidx.at[pl.ds(c * CHUNK, CHUNK)]

    def d_at(c):
        return didx.at[pl.ds(c * CHUNK, CHUNK)]

    for b in range(NBUF):
        pltpu.async_copy(gv.at[s_at(b)], bufs.at[b], gsems.at[b])

    @pl.loop(0, CHUNKS_PER_TILE, step=NBUF)
    def _(k):
        for b in range(NBUF):
            c = k + b
            pltpu.make_async_copy(gv.at[s_at(b)], bufs.at[b],
                                  gsems.at[b]).wait()
            pltpu.sync_copy(bufs.at[b], acc.at[d_at(c)], add=True)
            pltpu.async_copy(gv.at[s_at(c + NBUF)], bufs.at[b],
                             gsems.at[b])

    for b in range(NBUF):
        pltpu.make_async_copy(gv.at[s_at(b)], bufs.at[b], gsems.at[b]).wait()

    plsc.subcore_barrier()
    pltpu.sync_copy(acc.at[rows], out_hbm.at[cid].at[rows])


_sc_calls_cache = []


def _sc_calls():
    if not _sc_calls_cache:
        mesh = plsc.VectorSubcoreMesh(core_axis_name="c", subcore_axis_name="s",
                                      num_cores=NC, num_subcores=NS)
        cp = pltpu.CompilerParams(use_tc_tiling_on_sc=False)
        deg_call = pl.kernel(
            _deg_body,
            out_type=jax.ShapeDtypeStruct((NC, N_PAD, U_HID), jnp.float32),
            mesh=mesh,
            scratch_types=[
                pltpu.VMEM_SHARED((N_PAD, U_HID), jnp.float32),
                pltpu.VMEM((EDGES_PER_TILE,), jnp.int32),
                pltpu.VMEM((CHUNK, U_HID), jnp.float32),
                pltpu.SemaphoreType.DMA,
            ],
            compiler_params=cp,
        )
        scat_call = pl.kernel(
            _scat_body,
            out_type=jax.ShapeDtypeStruct((NC, N_PAD, U_HID), jnp.float32),
            mesh=mesh,
            scratch_types=[
                pltpu.VMEM_SHARED((N_PAD, U_HID), jnp.float32),
                pltpu.VMEM_SHARED((N_PAD, U_HID), jnp.float32),
                pltpu.VMEM((CHUNKS_SRC * CHUNK,), jnp.int32),
                pltpu.VMEM((EDGES_PER_TILE,), jnp.int32),
                pltpu.VMEM((NBUF, CHUNK, U_HID), jnp.float32),
                pltpu.SemaphoreType.DMA((NBUF,)),
                pltpu.SemaphoreType.DMA,
            ],
            compiler_params=cp,
        )
        _sc_calls_cache.append((deg_call, scat_call))
    return _sc_calls_cache[0]


def _matmul_body(x_ref, w_ref, hw_ref):
    hw_ref[...] = jnp.dot(x_ref[...], w_ref[...],
                          preferred_element_type=jnp.float32)


def _combine1_body(p_ref, hw_ref, dinv_ref, g_ref):
    deg = p_ref[0] + p_ref[1] + 1.0
    dinv = lax.rsqrt(jnp.maximum(deg, 1e-12))
    dinv_ref[...] = dinv
    g_ref[...] = dinv * hw_ref[...]


def _layer2_body(p_ref, g1_ref, dinv_ref, w2_ref, b1_ref, g2_ref):
    dinv = dinv_ref[...]
    h1 = jnp.maximum(dinv * (p_ref[0] + p_ref[1] + g1_ref[...]) + b1_ref[...],
                     0.0)
    hw2 = jnp.dot(h1, w2_ref[...], preferred_element_type=jnp.float32)
    g2_ref[...] = dinv * hw2


def _final_body(p_ref, g2_ref, dinv_ref, b2_ref, o_ref):
    o_ref[...] = jnp.maximum(
        dinv_ref[...] * (p_ref[0] + p_ref[1] + g2_ref[...]) + b2_ref[...], 0.0)


def kernel(x, edge_index, W1, b1, W2, b2):
    src = edge_index[0]
    dst = edge_index[1]
    pad_idx = jnp.full((PAD_E_TILE,), N_NODES, dtype=jnp.int32)
    x_pad = jnp.pad(x, ((0, N_PAD - N_NODES), (0, 0)))
    zeros = jnp.zeros((N_PAD, U_HID), jnp.float32)
    ones128 = jnp.ones((CHUNK, U_HID), jnp.float32)
    b1r = b1.reshape(1, U_HID)
    b2r = b2.reshape(1, U_HID)

    f32 = jnp.float32
    nu = jax.ShapeDtypeStruct((N_PAD, U_HID), f32)
    _deg_call, _scat_call = _sc_calls()

    hw1 = pl.pallas_call(_matmul_body, out_shape=nu)(x_pad, W1)
    deg_part = _deg_call(dst, pad_idx, ones128, zeros)
    dinv, g1 = pl.pallas_call(_combine1_body, out_shape=(nu, nu))(
        deg_part, hw1)
    part1 = _scat_call(g1, src, dst, pad_idx, zeros)
    g2 = pl.pallas_call(_layer2_body, out_shape=nu)(
        part1, g1, dinv, W2, b1r)
    part2 = _scat_call(g2, src, dst, pad_idx, zeros)
    out = pl.pallas_call(_final_body, out_shape=nu)(part2, g2, dinv, b2r)
    return out[:N_NODES]


# speedup vs baseline: 1.1522x; 1.0755x over previous
import jax
import jax.numpy as jnp
from jax import lax
from jax.experimental import pallas as pl
from jax.experimental.pallas import tpu as pltpu
from jax.experimental.pallas import tpu_sc as plsc

N_NODES = 10000
E_EDGES = 320000
D_IN = 128
U_HID = 16

NC = 2
NS = 16
N_TILES = NC * NS
CHUNK = 128
CHUNKS_PER_TILE = 80
EDGES_PER_TILE = CHUNKS_PER_TILE * CHUNK
E_PAD = N_TILES * EDGES_PER_TILE
N_PAD = 10240
ROWS_PER_TILE = N_PAD // NS
NBUF = 4
OVERRUN = NBUF * CHUNK
LAST_REAL = E_EDGES - (N_TILES - 1) * EDGES_PER_TILE
PAD_E_TILE = EDGES_PER_TILE - LAST_REAL
CHUNKS_SRC = CHUNKS_PER_TILE + NBUF


def _load_idx(flat_hbm, pad_hbm, idx, tile, sem):
    base = tile * EDGES_PER_TILE

    @pl.when(tile < N_TILES - 1)
    def _():
        pltpu.async_copy(flat_hbm.at[pl.ds(base, EDGES_PER_TILE)],
                         idx.at[pl.ds(0, EDGES_PER_TILE)], sem)

    @pl.when(tile == N_TILES - 1)
    def _():
        pltpu.async_copy(flat_hbm.at[pl.ds(base, LAST_REAL)],
                         idx.at[pl.ds(0, LAST_REAL)], sem)
        pltpu.async_copy(pad_hbm.at[pl.ds(0, PAD_E_TILE)],
                         idx.at[pl.ds(LAST_REAL, PAD_E_TILE)], sem)


def _wait_idx(flat_hbm, idx, sem):
    pltpu.make_async_copy(flat_hbm.at[pl.ds(0, EDGES_PER_TILE)],
                          idx.at[pl.ds(0, EDGES_PER_TILE)], sem).wait()


def _deg_body(edges_hbm, pad_hbm, ones_hbm, zeros_hbm, out_hbm,
              acc, didx, ones_v, sem):
    cid = lax.axis_index("c")
    sid = lax.axis_index("s")
    row0 = sid * ROWS_PER_TILE
    rows = pl.ds(row0, ROWS_PER_TILE)
    dst_hbm = edges_hbm.at[1]
    pltpu.async_copy(zeros_hbm.at[rows], acc.at[rows], sem)
    pltpu.async_copy(ones_hbm, ones_v, sem)
    tile = cid * NS + sid
    _load_idx(dst_hbm, pad_hbm, didx, tile, sem)
    pltpu.make_async_copy(zeros_hbm.at[rows], acc.at[rows], sem).wait()
    pltpu.make_async_copy(ones_hbm, ones_v, sem).wait()
    _wait_idx(dst_hbm, didx, sem)
    plsc.subcore_barrier()

    @pl.loop(0, CHUNKS_PER_TILE)
    def _(k):
        pltpu.async_copy(ones_v, acc.at[didx.at[pl.ds(k * CHUNK, CHUNK)]],
                         sem, add=True)

    @pl.loop(0, CHUNKS_PER_TILE)
    def _(k):
        pltpu.make_async_copy(ones_hbm, ones_v, sem).wait()

    plsc.subcore_barrier()
    pltpu.sync_copy(acc.at[rows], out_hbm.at[cid].at[rows])


def _scat_body(g_hbm, edges_hbm, pad_hbm, zeros_hbm, out_hbm,
               acc, gv, sidx, didx, bufs, gsems, ssem):
    cid = lax.axis_index("c")
    sid = lax.axis_index("s")
    src_hbm = edges_hbm.at[0]
    dst_hbm = edges_hbm.at[1]
    row0 = sid * ROWS_PER_TILE
    rows = pl.ds(row0, ROWS_PER_TILE)
    pltpu.async_copy(zeros_hbm.at[rows], acc.at[rows], ssem)
    pltpu.async_copy(g_hbm.at[rows], gv.at[rows], ssem)
    tile = cid * NS + sid
    _load_idx(src_hbm, pad_hbm, sidx, tile, ssem)
    pltpu.async_copy(pad_hbm.at[pl.ds(0, OVERRUN)],
                     sidx.at[pl.ds(EDGES_PER_TILE, OVERRUN)], ssem)
    _load_idx(dst_hbm, pad_hbm, didx, tile, ssem)
    pltpu.make_async_copy(zeros_hbm.at[rows], acc.at[rows], ssem).wait()
    pltpu.make_async_copy(g_hbm.at[rows], gv.at[rows], ssem).wait()
    _wait_idx(src_hbm, sidx, ssem)
    pltpu.make_async_copy(pad_hbm.at[pl.ds(0, OVERRUN)],
                          sidx.at[pl.ds(EDGES_PER_TILE, OVERRUN)],
                          ssem).wait()
    _wait_idx(dst_hbm, didx, ssem)
    plsc.subcore_barrier()

    def s_at(c):
        return sidx.at[pl.ds(c * CHUNK, CHUNK)]

    def d_at(c):
        return didx.at[pl.ds(c * CHUNK, CHUNK)]

    for b in range(NBUF):
        pltpu.async_copy(gv.at[s_at(b)], bufs.at[b], gsems.at[b])

    @pl.loop(0, CHUNKS_PER_TILE, step=NBUF)
    def _(k):
        for b in range(NBUF):
            c = k + b
            pltpu.make_async_copy(gv.at[s_at(b)], bufs.at[b],
                                  gsems.at[b]).wait()
            pltpu.sync_copy(bufs.at[b], acc.at[d_at(c)], add=True)
            pltpu.async_copy(gv.at[s_at(c + NBUF)], bufs.at[b],
                             gsems.at[b])

    for b in range(NBUF):
        pltpu.make_async_copy(gv.at[s_at(b)], bufs.at[b], gsems.at[b]).wait()

    plsc.subcore_barrier()
    pltpu.sync_copy(acc.at[rows], out_hbm.at[cid].at[rows])


_sc_calls_cache = []


def _sc_calls():
    if not _sc_calls_cache:
        mesh = plsc.VectorSubcoreMesh(core_axis_name="c", subcore_axis_name="s",
                                      num_cores=NC, num_subcores=NS)
        cp = pltpu.CompilerParams(use_tc_tiling_on_sc=False)
        deg_call = pl.kernel(
            _deg_body,
            out_type=jax.ShapeDtypeStruct((NC, N_PAD, U_HID), jnp.float32),
            mesh=mesh,
            scratch_types=[
                pltpu.VMEM_SHARED((N_PAD, U_HID), jnp.float32),
                pltpu.VMEM((EDGES_PER_TILE,), jnp.int32),
                pltpu.VMEM((CHUNK, U_HID), jnp.float32),
                pltpu.SemaphoreType.DMA,
            ],
            compiler_params=cp,
        )
        scat_call = pl.kernel(
            _scat_body,
            out_type=jax.ShapeDtypeStruct((NC, N_PAD, U_HID), jnp.float32),
            mesh=mesh,
            scratch_types=[
                pltpu.VMEM_SHARED((N_PAD, U_HID), jnp.float32),
                pltpu.VMEM_SHARED((N_PAD, U_HID), jnp.float32),
                pltpu.VMEM((CHUNKS_SRC * CHUNK,), jnp.int32),
                pltpu.VMEM((EDGES_PER_TILE,), jnp.int32),
                pltpu.VMEM((NBUF, CHUNK, U_HID), jnp.float32),
                pltpu.SemaphoreType.DMA((NBUF,)),
                pltpu.SemaphoreType.DMA,
            ],
            compiler_params=cp,
        )
        _sc_calls_cache.append((deg_call, scat_call))
    return _sc_calls_cache[0]


def _matmul_body(x_ref, w_ref, hw_ref):
    hw_ref[...] = jnp.dot(x_ref[...], w_ref[...],
                          preferred_element_type=jnp.float32)


def _combine1_body(p_ref, hw_ref, dinv_ref, g_ref):
    deg = p_ref[0] + p_ref[1] + 1.0
    dinv = lax.rsqrt(jnp.maximum(deg, 1e-12))
    dinv_ref[...] = dinv
    g_ref[...] = dinv * hw_ref[...]


def _layer2_body(p_ref, g1_ref, dinv_ref, w2_ref, b1_ref, g2_ref):
    dinv = dinv_ref[...]
    h1 = jnp.maximum(dinv * (p_ref[0] + p_ref[1] + g1_ref[...]) + b1_ref[...],
                     0.0)
    hw2 = jnp.dot(h1, w2_ref[...], preferred_element_type=jnp.float32)
    g2_ref[...] = dinv * hw2


def _final_body(p_ref, g2_ref, dinv_ref, b2_ref, o_ref):
    v = jnp.maximum(
        dinv_ref[...] * (p_ref[0] + p_ref[1] + g2_ref[...]) + b2_ref[...], 0.0)
    o_ref[...] = v[:N_NODES]


def kernel(x, edge_index, W1, b1, W2, b2):
    pad_idx = jnp.full((PAD_E_TILE,), N_NODES, dtype=jnp.int32)
    x_pad = jnp.pad(x, ((0, N_PAD - N_NODES), (0, 0)))
    zeros = jnp.zeros((N_PAD, U_HID), jnp.float32)
    ones128 = jnp.ones((CHUNK, U_HID), jnp.float32)
    b1r = b1.reshape(1, U_HID)
    b2r = b2.reshape(1, U_HID)

    f32 = jnp.float32
    nu = jax.ShapeDtypeStruct((N_PAD, U_HID), f32)
    _deg_call, _scat_call = _sc_calls()

    hw1 = pl.pallas_call(_matmul_body, out_shape=nu)(x_pad, W1)
    deg_part = _deg_call(edge_index, pad_idx, ones128, zeros)
    dinv, g1 = pl.pallas_call(_combine1_body, out_shape=(nu, nu))(
        deg_part, hw1)
    part1 = _scat_call(g1, edge_index, pad_idx, zeros)
    g2 = pl.pallas_call(_layer2_body, out_shape=nu)(
        part1, g1, dinv, W2, b1r)
    part2 = _scat_call(g2, edge_index, pad_idx, zeros)
    out = pl.pallas_call(
        _final_body,
        out_shape=jax.ShapeDtypeStruct((N_NODES, U_HID), f32))(
        part2, g2, dinv, b2r)
    return out
